# baseline (speedup 1.0000x reference)
import functools

import jax
import jax.numpy as jnp
from jax import lax
from jax.experimental import pallas as pl
from jax.experimental.pallas import tpu as pltpu
from jax.experimental.pallas import tpu_sc as plsc

BATCH = 4096
HIST = 200
EMBED_DIM = 64
SIZE = 128

N = BATCH * HIST
GROW = 128
NW = 32
IDX_ROWS = N // GROW
ROWS_PER_W = IDX_ROWS // NW


def _sc_gather(table, idx2d):
    mesh = plsc.VectorSubcoreMesh(
        core_axis_name="c", subcore_axis_name="s", num_cores=2, num_subcores=16
    )

    @functools.partial(
        pl.kernel,
        out_type=jax.ShapeDtypeStruct((N, EMBED_DIM), jnp.float32),
        mesh=mesh,
        scratch_types=[
            pltpu.VMEM((ROWS_PER_W, GROW), jnp.int32),
            pltpu.VMEM((GROW, EMBED_DIM), jnp.float32),
            pltpu.SemaphoreType.DMA,
        ],
        compiler_params=pltpu.CompilerParams(use_tc_tiling_on_sc=False),
    )
    def k(table_hbm, idx_hbm, emb_hbm, idx_v, rows_v, sem):
        wid = lax.axis_index("s") * 2 + lax.axis_index("c")
        base = wid * ROWS_PER_W
        pltpu.sync_copy(idx_hbm.at[pl.ds(base, ROWS_PER_W)], idx_v)

        def body(j, carry):
            pltpu.async_copy(table_hbm.at[idx_v.at[j]], rows_v, sem).wait()
            pltpu.sync_copy(rows_v, emb_hbm.at[pl.ds((base + j) * GROW, GROW)])
            return carry

        lax.fori_loop(0, ROWS_PER_W, body, 0)

    return k(table, idx2d)


def _tc_project(emb, W):
    BM = 8192

    def mm(emb_ref, w_ref, out_ref):
        out_ref[...] = lax.dot_general(
            emb_ref[...], w_ref[...],
            (((1,), (1,)), ((), ())),
            preferred_element_type=jnp.float32,
        )

    return pl.pallas_call(
        mm,
        grid=(N // BM,),
        in_specs=[
            pl.BlockSpec((BM, EMBED_DIM), lambda i: (i, 0)),
            pl.BlockSpec((SIZE, EMBED_DIM), lambda i: (0, 0)),
        ],
        out_specs=pl.BlockSpec((BM, SIZE), lambda i: (i, 0)),
        out_shape=jax.ShapeDtypeStruct((N, SIZE), jnp.float32),
    )(emb, W)


def kernel(input, table, W):
    idx2d = input.reshape(IDX_ROWS, GROW).astype(jnp.int32)
    emb = _sc_gather(table, idx2d)
    out = _tc_project(emb, W)
    return out.reshape(BATCH, HIST, SIZE)


# speedup vs baseline: 1.5091x; 1.5091x over previous
import functools

import jax
import jax.numpy as jnp
from jax import lax
from jax.experimental import pallas as pl
from jax.experimental.pallas import tpu as pltpu
from jax.experimental.pallas import tpu_sc as plsc

BATCH = 4096
HIST = 200
EMBED_DIM = 64
SIZE = 128
NUM_EMB = 1000000

N = BATCH * HIST
GROW = 128
NW = 32
IDX_ROWS = N // GROW
ROWS_PER_W = IDX_ROWS // NW


def _tc_project_table(table, W):
    BT = 20000

    def mm(t_ref, w_ref, p_ref):
        p_ref[...] = lax.dot_general(
            t_ref[...], w_ref[...],
            (((1,), (1,)), ((), ())),
            preferred_element_type=jnp.float32,
        )

    return pl.pallas_call(
        mm,
        grid=(NUM_EMB // BT,),
        in_specs=[
            pl.BlockSpec((BT, EMBED_DIM), lambda i: (i, 0)),
            pl.BlockSpec((SIZE, EMBED_DIM), lambda i: (0, 0)),
        ],
        out_specs=pl.BlockSpec((BT, SIZE), lambda i: (i, 0)),
        out_shape=jax.ShapeDtypeStruct((NUM_EMB, SIZE), jnp.float32),
    )(table, W)


def _sc_gather(P, idx2d):
    mesh = plsc.VectorSubcoreMesh(
        core_axis_name="c", subcore_axis_name="s", num_cores=2, num_subcores=16
    )

    @functools.partial(
        pl.kernel,
        out_type=jax.ShapeDtypeStruct((N, SIZE), jnp.float32),
        mesh=mesh,
        scratch_types=[
            pltpu.VMEM((ROWS_PER_W, GROW), jnp.int32),
            pltpu.VMEM((GROW, SIZE), jnp.float32),
            pltpu.VMEM((GROW, SIZE), jnp.float32),
            pltpu.SemaphoreType.DMA,
            pltpu.SemaphoreType.DMA,
        ],
        compiler_params=pltpu.CompilerParams(use_tc_tiling_on_sc=True),
    )
    def k(p_hbm, idx_hbm, out_hbm, idx_v, rows0, rows1, sem0, sem1):
        wid = lax.axis_index("s") * 2 + lax.axis_index("c")
        base = wid * ROWS_PER_W
        pltpu.sync_copy(idx_hbm.at[pl.ds(base, ROWS_PER_W)], idx_v)

        pltpu.make_async_copy(p_hbm.at[idx_v.at[0]], rows0, sem0).start()

        def body(t, carry):
            j0 = t * 2
            j1 = j0 + 1
            pltpu.make_async_copy(p_hbm.at[idx_v.at[j1]], rows1, sem1).start()
            pltpu.make_async_copy(p_hbm.at[idx_v.at[j0]], rows0, sem0).wait()
            pltpu.sync_copy(rows0, out_hbm.at[pl.ds((base + j0) * GROW, GROW)])

            @pl.when(t + 1 < ROWS_PER_W // 2)
            def _():
                pltpu.make_async_copy(
                    p_hbm.at[idx_v.at[j0 + 2]], rows0, sem0).start()

            pltpu.make_async_copy(p_hbm.at[idx_v.at[j1]], rows1, sem1).wait()
            pltpu.sync_copy(rows1, out_hbm.at[pl.ds((base + j1) * GROW, GROW)])
            return carry

        lax.fori_loop(0, ROWS_PER_W // 2, body, 0)

    return k(P, idx2d)


def kernel(input, table, W):
    idx2d = input.reshape(IDX_ROWS, GROW).astype(jnp.int32)
    P = _tc_project_table(table, W)
    out = _sc_gather(P, idx2d)
    return out.reshape(BATCH, HIST, SIZE)


# speedup vs baseline: 2.6136x; 1.7319x over previous
import functools

import jax
import jax.numpy as jnp
from jax import lax
from jax.experimental import pallas as pl
from jax.experimental.pallas import tpu as pltpu
from jax.experimental.pallas import tpu_sc as plsc

BATCH = 4096
HIST = 200
EMBED_DIM = 64
SIZE = 128
NUM_EMB = 1000000

N = BATCH * HIST
GROW = 128
NW = 32
IDX_ROWS = N // GROW
ROWS_PER_W = IDX_ROWS // NW


def _tc_project_table(tableT, WT):
    BT = 16384

    def mm(t_ref, w_ref, p_ref):
        p_ref[...] = lax.dot_general(
            t_ref[...], w_ref[...],
            (((0,), (0,)), ((), ())),
            preferred_element_type=jnp.float32,
        )

    return pl.pallas_call(
        mm,
        grid=((NUM_EMB + BT - 1) // BT,),
        in_specs=[
            pl.BlockSpec((EMBED_DIM, BT), lambda i: (0, i)),
            pl.BlockSpec((EMBED_DIM, SIZE), lambda i: (0, 0)),
        ],
        out_specs=pl.BlockSpec((BT, SIZE), lambda i: (i, 0)),
        out_shape=jax.ShapeDtypeStruct((NUM_EMB, SIZE), jnp.float32),
    )(tableT, WT)


def _sc_gather(P, idx2d):
    mesh = plsc.VectorSubcoreMesh(
        core_axis_name="c", subcore_axis_name="s", num_cores=2, num_subcores=16
    )

    @functools.partial(
        pl.kernel,
        out_type=jax.ShapeDtypeStruct((N, SIZE), jnp.float32),
        mesh=mesh,
        scratch_types=[
            pltpu.VMEM((ROWS_PER_W, GROW), jnp.int32),
            pltpu.VMEM((GROW, SIZE), jnp.float32),
            pltpu.VMEM((GROW, SIZE), jnp.float32),
            pltpu.SemaphoreType.DMA,
            pltpu.SemaphoreType.DMA,
        ],
        compiler_params=pltpu.CompilerParams(use_tc_tiling_on_sc=True),
    )
    def k(p_hbm, idx_hbm, out_hbm, idx_v, rows0, rows1, sem0, sem1):
        wid = lax.axis_index("s") * 2 + lax.axis_index("c")
        base = wid * ROWS_PER_W
        pltpu.sync_copy(idx_hbm.at[pl.ds(base, ROWS_PER_W)], idx_v)

        pltpu.make_async_copy(p_hbm.at[idx_v.at[0]], rows0, sem0).start()

        def body(t, carry):
            j0 = t * 2
            j1 = j0 + 1
            pltpu.make_async_copy(p_hbm.at[idx_v.at[j1]], rows1, sem1).start()
            pltpu.make_async_copy(p_hbm.at[idx_v.at[j0]], rows0, sem0).wait()
            pltpu.sync_copy(rows0, out_hbm.at[pl.ds((base + j0) * GROW, GROW)])

            @pl.when(t + 1 < ROWS_PER_W // 2)
            def _():
                pltpu.make_async_copy(
                    p_hbm.at[idx_v.at[j0 + 2]], rows0, sem0).start()

            pltpu.make_async_copy(p_hbm.at[idx_v.at[j1]], rows1, sem1).wait()
            pltpu.sync_copy(rows1, out_hbm.at[pl.ds((base + j1) * GROW, GROW)])
            return carry

        lax.fori_loop(0, ROWS_PER_W // 2, body, 0)

    return k(P, idx2d)


def kernel(input, table, W):
    idx2d = input.reshape(IDX_ROWS, GROW).astype(jnp.int32)
    P = _tc_project_table(table.T, W.T)
    out = _sc_gather(P, idx2d)
    return out.reshape(BATCH, HIST, SIZE)
